# baseline (speedup 1.0000x reference)
import functools

import jax
import jax.numpy as jnp
from jax import lax
from jax.experimental import pallas as pl
from jax.experimental.pallas import tpu as pltpu
from jax.experimental.pallas import tpu_sc as plsc

BATCH = 16384
D = 64
NC = 2
NS = 16
L = 16
NW = NC * NS
BPW = BATCH // NW
CHUNK = 256
NCHUNK = BPW // CHUNK

_mesh = plsc.VectorSubcoreMesh(core_axis_name="c", subcore_axis_name="s")


@functools.partial(
    pl.kernel,
    mesh=_mesh,
    out_type=jax.ShapeDtypeStruct((D, BATCH), jnp.float32),
    compiler_params=pltpu.CompilerParams(needs_layout_passes=False),
    scratch_types=[
        pltpu.VMEM((BPW,), jnp.int32),
        pltpu.VMEM((CHUNK, 2 * D), jnp.float32),
        pltpu.VMEM((CHUNK, 2 * D), jnp.float32),
        pltpu.VMEM((D, CHUNK), jnp.float32),
        pltpu.VMEM((D, CHUNK), jnp.float32),
        pltpu.SemaphoreType.DMA,
        pltpu.SemaphoreType.DMA,
        pltpu.SemaphoreType.DMA,
        pltpu.SemaphoreType.DMA,
        pltpu.SemaphoreType.DMA,
        pltpu.SemaphoreType.DMA,
    ],
)
def _film(xt_hbm, idx_hbm, film_hbm, out_hbm,
          idx_v, rows0, rows1, xt0, xt1,
          gs0, gs1, xs0, xs1, os0, os1):
    rows = (rows0, rows1)
    xt = (xt0, xt1)
    gsem = (gs0, gs1)
    xsem = (xs0, xs1)
    osem = (os0, os1)

    wid = lax.axis_index("s") * NC + lax.axis_index("c")
    base = wid * BPW
    pltpu.sync_copy(idx_hbm.at[pl.ds(base, BPW)], idx_v)

    gathers = [None, None]
    xcopies = [None, None]
    ostores = [None, None]

    def start(c):
        b = c % 2
        gathers[b] = pltpu.async_copy(
            film_hbm.at[idx_v.at[pl.ds(c * CHUNK, CHUNK)]], rows[b], gsem[b])
        xcopies[b] = pltpu.async_copy(
            xt_hbm.at[:, pl.ds(base + c * CHUNK, CHUNK)], xt[b], xsem[b])

    start(0)
    for c in range(NCHUNK):
        b = c % 2
        if c + 1 < NCHUNK:
            nb = (c + 1) % 2
            if c >= 1:
                ostores[nb].wait()
            start(c + 1)
        gathers[b].wait()
        xcopies[b].wait()

        @plsc.parallel_loop(0, CHUNK // L)
        def body(rg):
            row_ids = lax.iota(jnp.int32, L) + rg * L
            for col in range(D):
                g = plsc.load_gather(
                    rows[b], [row_ids, jnp.full((L,), col, jnp.int32)])
                bta = plsc.load_gather(
                    rows[b], [row_ids, jnp.full((L,), D + col, jnp.int32)])
                sl = pl.ds(rg * L, L)
                xt[b][col, sl] = g * xt[b][col, sl] + bta

        ostores[b] = pltpu.async_copy(
            xt[b], out_hbm.at[:, pl.ds(base + c * CHUNK, CHUNK)], osem[b])

    ostores[0].wait()
    ostores[1].wait()


def kernel(x, cell_line, film):
    out_t = _film(x.T, cell_line, film)
    return (out_t.T, cell_line)


# speedup vs baseline: 1.2268x; 1.2268x over previous
import functools

import jax
import jax.numpy as jnp
from jax import lax
from jax.experimental import pallas as pl
from jax.experimental.pallas import tpu as pltpu
from jax.experimental.pallas import tpu_sc as plsc

BATCH = 16384
D = 64
NC = 2
NS = 16
L = 16
NW = NC * NS
BPW = BATCH // NW
CHUNK = 256
NCHUNK = BPW // CHUNK
XSTRIDE = CHUNK + 1

_mesh = plsc.VectorSubcoreMesh(core_axis_name="c", subcore_axis_name="s")


@functools.partial(
    pl.kernel,
    mesh=_mesh,
    out_type=jax.ShapeDtypeStruct((D, BATCH), jnp.float32),
    compiler_params=pltpu.CompilerParams(needs_layout_passes=False),
    scratch_types=[
        pltpu.VMEM((BPW,), jnp.int32),
        pltpu.VMEM((CHUNK, 2 * D), jnp.float32),
        pltpu.VMEM((CHUNK, 2 * D), jnp.float32),
        pltpu.VMEM((D, XSTRIDE), jnp.float32),
        pltpu.VMEM((D, XSTRIDE), jnp.float32),
        pltpu.SemaphoreType.DMA,
        pltpu.SemaphoreType.DMA,
        pltpu.SemaphoreType.DMA,
        pltpu.SemaphoreType.DMA,
        pltpu.SemaphoreType.DMA,
        pltpu.SemaphoreType.DMA,
    ],
)
def _film(xt_hbm, idx_hbm, film_hbm, out_hbm,
          idx_v, rows0, rows1, xt0, xt1,
          gs0, gs1, xs0, xs1, os0, os1):
    rows = (rows0, rows1)
    xt = (xt0, xt1)
    gsem = (gs0, gs1)
    xsem = (xs0, xs1)
    osem = (os0, os1)

    wid = lax.axis_index("s") * NC + lax.axis_index("c")
    base = wid * BPW
    pltpu.sync_copy(idx_hbm.at[pl.ds(base, BPW)], idx_v)

    gathers = [None, None]
    xcopies = [None, None]
    ostores = [None, None]

    def start(c):
        b = c % 2
        gathers[b] = pltpu.async_copy(
            film_hbm.at[idx_v.at[pl.ds(c * CHUNK, CHUNK)]], rows[b], gsem[b])
        xcopies[b] = pltpu.async_copy(
            xt_hbm.at[:, pl.ds(base + c * CHUNK, CHUNK)],
            xt[b].at[:, pl.ds(0, CHUNK)], xsem[b])

    start(0)
    for c in range(NCHUNK):
        b = c % 2
        if c + 1 < NCHUNK:
            nb = (c + 1) % 2
            if c >= 1:
                ostores[nb].wait()
            start(c + 1)
        gathers[b].wait()
        xcopies[b].wait()

        col_ids = [lax.iota(jnp.int32, L) + g * L for g in range(D // L)]

        @plsc.parallel_loop(0, CHUNK, unroll=2)
        def body(r):
            r_vec = jnp.full((L,), 0, jnp.int32) + r
            for g in range(D // L):
                gam = rows[b][r, pl.ds(g * L, L)]
                bta = rows[b][r, pl.ds(D + g * L, L)]
                xv = plsc.load_gather(xt[b], [col_ids[g], r_vec])
                plsc.store_scatter(xt[b], [col_ids[g], r_vec], gam * xv + bta)

        ostores[b] = pltpu.async_copy(
            xt[b].at[:, pl.ds(0, CHUNK)],
            out_hbm.at[:, pl.ds(base + c * CHUNK, CHUNK)], osem[b])

    ostores[0].wait()
    ostores[1].wait()


def kernel(x, cell_line, film):
    out_t = _film(x.T, cell_line, film)
    return (out_t.T, cell_line)


# speedup vs baseline: 1.2535x; 1.0218x over previous
import functools

import jax
import jax.numpy as jnp
from jax import lax
from jax.experimental import pallas as pl
from jax.experimental.pallas import tpu as pltpu
from jax.experimental.pallas import tpu_sc as plsc

BATCH = 16384
D = 64
NC = 2
NS = 16
L = 16
NW = NC * NS
BPW = BATCH // NW
CHUNK = 128
NCHUNK = BPW // CHUNK
XSTRIDE = 136

_mesh = plsc.VectorSubcoreMesh(core_axis_name="c", subcore_axis_name="s")


@functools.partial(
    pl.kernel,
    mesh=_mesh,
    out_type=jax.ShapeDtypeStruct((D, BATCH), jnp.float32),
    compiler_params=pltpu.CompilerParams(needs_layout_passes=False),
    scratch_types=[
        pltpu.VMEM((BPW,), jnp.int32),
        pltpu.VMEM((CHUNK, 2 * D), jnp.float32),
        pltpu.VMEM((CHUNK, 2 * D), jnp.float32),
        pltpu.VMEM((D, XSTRIDE), jnp.float32),
        pltpu.VMEM((D, XSTRIDE), jnp.float32),
        pltpu.VMEM((D, XSTRIDE), jnp.float32),
        pltpu.VMEM((D, XSTRIDE), jnp.float32),
        pltpu.SemaphoreType.DMA,
        pltpu.SemaphoreType.DMA,
        pltpu.SemaphoreType.DMA,
        pltpu.SemaphoreType.DMA,
        pltpu.SemaphoreType.DMA,
        pltpu.SemaphoreType.DMA,
    ],
)
def _film(xt_hbm, idx_hbm, film_hbm, out_hbm,
          idx_v, rows0, rows1, xt0, xt1, ot0, ot1,
          gs0, gs1, xs0, xs1, os0, os1):
    rows = (rows0, rows1)
    xt = (xt0, xt1)
    ot = (ot0, ot1)
    gsem = (gs0, gs1)
    xsem = (xs0, xs1)
    osem = (os0, os1)

    wid = lax.axis_index("s") * NC + lax.axis_index("c")
    base = wid * BPW
    pltpu.sync_copy(idx_hbm.at[pl.ds(base, BPW)], idx_v)

    gathers = [None, None]
    xcopies = [None, None]
    ostores = [None, None]

    def start(c):
        b = c % 2
        gathers[b] = pltpu.async_copy(
            film_hbm.at[idx_v.at[pl.ds(c * CHUNK, CHUNK)]], rows[b], gsem[b])
        xcopies[b] = pltpu.async_copy(
            xt_hbm.at[:, pl.ds(base + c * CHUNK, CHUNK)],
            xt[b].at[:, pl.ds(0, CHUNK)], xsem[b])

    start(0)
    for c in range(NCHUNK):
        b = c % 2
        if c + 1 < NCHUNK:
            nb = (c + 1) % 2
            if c >= 2:
                ostores[nb].wait()
            start(c + 1)
        gathers[b].wait()
        xcopies[b].wait()

        col_ids = [lax.iota(jnp.int32, L) + g * L for g in range(D // L)]

        @plsc.parallel_loop(0, CHUNK, unroll=4)
        def body(r):
            r_vec = jnp.zeros((L,), jnp.int32) + r
            for g in range(D // L):
                gam = rows[b][r, pl.ds(g * L, L)]
                bta = rows[b][r, pl.ds(D + g * L, L)]
                xv = plsc.load_gather(xt[b], [col_ids[g], r_vec])
                plsc.store_scatter(ot[b], [col_ids[g], r_vec], gam * xv + bta)

        ostores[b] = pltpu.async_copy(
            ot[b].at[:, pl.ds(0, CHUNK)],
            out_hbm.at[:, pl.ds(base + c * CHUNK, CHUNK)], osem[b])

    ostores[0].wait()
    ostores[1].wait()


def kernel(x, cell_line, film):
    out_t = _film(x.T, cell_line, film)
    return (out_t.T, cell_line)


# speedup vs baseline: 1.2801x; 1.0212x over previous
import functools

import jax
import jax.numpy as jnp
from jax import lax
from jax.experimental import pallas as pl
from jax.experimental.pallas import tpu as pltpu
from jax.experimental.pallas import tpu_sc as plsc

BATCH = 16384
D = 64
NC = 2
NS = 16
NW = NC * NS
BPW = BATCH // NW
CHUNK = 256
NCHUNK = BPW // CHUNK
BLK = 512

_mesh = plsc.VectorSubcoreMesh(core_axis_name="c", subcore_axis_name="s")


@functools.partial(
    pl.kernel,
    mesh=_mesh,
    out_type=jax.ShapeDtypeStruct((BATCH, 2 * D), jnp.float32),
    scratch_types=[
        pltpu.VMEM((BPW,), jnp.int32),
        pltpu.VMEM((CHUNK, 2 * D), jnp.float32),
        pltpu.VMEM((CHUNK, 2 * D), jnp.float32),
        pltpu.SemaphoreType.DMA,
        pltpu.SemaphoreType.DMA,
        pltpu.SemaphoreType.DMA,
        pltpu.SemaphoreType.DMA,
    ],
)
def _gather(idx_hbm, film_hbm, g_hbm, idx_v, rows0, rows1,
            gs0, gs1, os0, os1):
    rows = (rows0, rows1)
    gsem = (gs0, gs1)
    osem = (os0, os1)

    wid = lax.axis_index("s") * NC + lax.axis_index("c")
    base = wid * BPW
    pltpu.sync_copy(idx_hbm.at[pl.ds(base, BPW)], idx_v)

    gathers = [None, None]
    ostores = [None, None]

    def start(c):
        b = c % 2
        gathers[b] = pltpu.async_copy(
            film_hbm.at[idx_v.at[pl.ds(c * CHUNK, CHUNK)]], rows[b], gsem[b])

    start(0)
    for c in range(NCHUNK):
        b = c % 2
        gathers[b].wait()
        if c + 1 < NCHUNK:
            nb = (c + 1) % 2
            if c >= 1:
                ostores[nb].wait()
            start(c + 1)
        ostores[b] = pltpu.async_copy(
            rows[b], g_hbm.at[pl.ds(base + c * CHUNK, CHUNK)], osem[b])

    ostores[(NCHUNK - 2) % 2].wait()
    ostores[(NCHUNK - 1) % 2].wait()


def _affine_body(g_ref, xt_ref, o_ref):
    gt = g_ref[...].T
    o_ref[...] = gt[:D, :] * xt_ref[...] + gt[D:, :]


def _affine(g, xt):
    return pl.pallas_call(
        _affine_body,
        out_shape=jax.ShapeDtypeStruct((D, BATCH), jnp.float32),
        grid=(BATCH // BLK,),
        in_specs=[
            pl.BlockSpec((BLK, 2 * D), lambda i: (i, 0)),
            pl.BlockSpec((D, BLK), lambda i: (0, i)),
        ],
        out_specs=pl.BlockSpec((D, BLK), lambda i: (0, i)),
    )(g, xt)


def kernel(x, cell_line, film):
    g = _gather(cell_line, film)
    out_t = _affine(g, x.T)
    return (out_t.T, cell_line)


# speedup vs baseline: 1.3618x; 1.0638x over previous
import functools

import jax
import jax.numpy as jnp
from jax import lax
from jax.experimental import pallas as pl
from jax.experimental.pallas import tpu as pltpu
from jax.experimental.pallas import tpu_sc as plsc

BATCH = 16384
D = 64
NC = 2
NS = 16
L = 16
NW = NC * NS
BPW = BATCH // NW
CHUNK = 128
NCHUNK = BPW // CHUNK

_mesh = plsc.VectorSubcoreMesh(core_axis_name="c", subcore_axis_name="s")


@functools.partial(
    pl.kernel,
    mesh=_mesh,
    out_type=jax.ShapeDtypeStruct((BATCH, D), jnp.float32),
    scratch_types=[
        pltpu.VMEM((NCHUNK, CHUNK), jnp.int32),
        pltpu.VMEM((CHUNK, 2 * D), jnp.float32),
        pltpu.VMEM((CHUNK, 2 * D), jnp.float32),
        pltpu.VMEM((CHUNK, D), jnp.float32),
        pltpu.VMEM((CHUNK, D), jnp.float32),
        pltpu.SemaphoreType.DMA,
        pltpu.SemaphoreType.DMA,
        pltpu.SemaphoreType.DMA,
        pltpu.SemaphoreType.DMA,
        pltpu.SemaphoreType.DMA,
        pltpu.SemaphoreType.DMA,
    ],
)
def _film(x_hbm, idx_hbm, film_hbm, out_hbm,
          idx_v, rows0, rows1, xb0, xb1,
          gs0, gs1, xs0, xs1, os0, os1):
    rows = (rows0, rows1)
    xb = (xb0, xb1)
    gsem = (gs0, gs1)
    xsem = (xs0, xs1)
    osem = (os0, os1)

    wid = lax.axis_index("s") * NC + lax.axis_index("c")
    base = wid * BPW
    pltpu.sync_copy(idx_hbm.at[wid], idx_v)

    gathers = [None, None]
    xcopies = [None, None]
    ostores = [None, None]

    def start(c):
        b = c % 2
        gathers[b] = pltpu.async_copy(film_hbm.at[idx_v.at[c]], rows[b], gsem[b])
        xcopies[b] = pltpu.async_copy(
            x_hbm.at[pl.ds(base + c * CHUNK, CHUNK)], xb[b], xsem[b])

    start(0)
    for c in range(NCHUNK):
        b = c % 2
        if c + 1 < NCHUNK:
            nb = (c + 1) % 2
            if c >= 1:
                ostores[nb].wait()
            start(c + 1)
        gathers[b].wait()
        xcopies[b].wait()

        @plsc.parallel_loop(0, CHUNK, unroll=16)
        def body(r):
            for j in range(D // L):
                sl = pl.ds(j * L, L)
                xb[b][r, sl] = rows[b][r, sl] * xb[b][r, sl] \
                    + rows[b][r, pl.ds(D + j * L, L)]

        ostores[b] = pltpu.async_copy(
            xb[b], out_hbm.at[pl.ds(base + c * CHUNK, CHUNK)], osem[b])

    ostores[(NCHUNK - 2) % 2].wait()
    ostores[(NCHUNK - 1) % 2].wait()


def kernel(x, cell_line, film):
    idx = cell_line.reshape(NW, NCHUNK, CHUNK)
    out = _film(x, idx, film)
    return (out, cell_line)


# speedup vs baseline: 1.4175x; 1.0409x over previous
import functools

import jax
import jax.numpy as jnp
from jax import lax
from jax.experimental import pallas as pl
from jax.experimental.pallas import tpu as pltpu
from jax.experimental.pallas import tpu_sc as plsc

BATCH = 16384
D = 64
NC = 2
NS = 16
L = 16
NW = NC * NS
BPW = BATCH // NW
CHUNK = 128
NCHUNK = BPW // CHUNK

_mesh = plsc.VectorSubcoreMesh(core_axis_name="c", subcore_axis_name="s")


@functools.partial(
    pl.kernel,
    mesh=_mesh,
    out_type=jax.ShapeDtypeStruct((BATCH, D), jnp.float32),
    scratch_types=[
        pltpu.VMEM((NCHUNK, CHUNK), jnp.int32),
        pltpu.VMEM((CHUNK, 2 * D), jnp.float32),
        pltpu.VMEM((CHUNK, 2 * D), jnp.float32),
        pltpu.VMEM((CHUNK, D), jnp.float32),
        pltpu.VMEM((CHUNK, D), jnp.float32),
        pltpu.VMEM((CHUNK, D), jnp.float32),
        pltpu.VMEM((CHUNK, D), jnp.float32),
        pltpu.SemaphoreType.DMA,
        pltpu.SemaphoreType.DMA,
        pltpu.SemaphoreType.DMA,
        pltpu.SemaphoreType.DMA,
        pltpu.SemaphoreType.DMA,
        pltpu.SemaphoreType.DMA,
    ],
)
def _film(x_hbm, idx_hbm, film_hbm, out_hbm,
          idx_v, rows0, rows1, xb0, xb1, ob0, ob1,
          gs0, gs1, xs0, xs1, os0, os1):
    rows = (rows0, rows1)
    xb = (xb0, xb1)
    ob = (ob0, ob1)
    gsem = (gs0, gs1)
    xsem = (xs0, xs1)
    osem = (os0, os1)

    wid = lax.axis_index("s") * NC + lax.axis_index("c")
    base = wid * BPW
    pltpu.sync_copy(idx_hbm.at[wid], idx_v)

    gathers = [None, None]
    xcopies = [None, None]
    ostores = [None, None]

    def start(c):
        b = c % 2
        gathers[b] = pltpu.async_copy(film_hbm.at[idx_v.at[c]], rows[b], gsem[b])
        xcopies[b] = pltpu.async_copy(
            x_hbm.at[pl.ds(base + c * CHUNK, CHUNK)], xb[b], xsem[b])

    start(0)
    for c in range(NCHUNK):
        b = c % 2
        if c + 1 < NCHUNK:
            nb = (c + 1) % 2
            start(c + 1)
        gathers[b].wait()
        xcopies[b].wait()
        if c >= 2:
            ostores[b].wait()

        @plsc.parallel_loop(0, CHUNK, unroll=8)
        def body(r):
            for j in range(D // L):
                sl = pl.ds(j * L, L)
                ob[b][r, sl] = rows[b][r, sl] * xb[b][r, sl] \
                    + rows[b][r, pl.ds(D + j * L, L)]

        ostores[b] = pltpu.async_copy(
            ob[b], out_hbm.at[pl.ds(base + c * CHUNK, CHUNK)], osem[b])

    ostores[(NCHUNK - 2) % 2].wait()
    ostores[(NCHUNK - 1) % 2].wait()


def kernel(x, cell_line, film):
    idx = cell_line.reshape(NW, NCHUNK, CHUNK)
    out = _film(x, idx, film)
    return (out, cell_line)
